# baseline (speedup 1.0000x reference)
import functools

import jax
import jax.numpy as jnp
from jax import lax
from jax.experimental import pallas as pl
from jax.experimental.pallas import tpu as pltpu
from jax.experimental.pallas import tpu_sc as plsc

N_FIELDS = 26
VOCAB = 100000
EMB = 16
OHE = 100
HID = 128
BATCH = 16384
EMB_FEAT = N_FIELDS * EMB
NPLANE = 4

TOT_ROWS = BATCH * N_FIELDS
NC, NS = 2, 16
NW = NC * NS
BATCH_PER_W = BATCH // NW
SUPER = 64
NSUPER = BATCH_PER_W // SUPER
GROUP = 4
GIDX = GROUP * N_FIELDS
GPS = SUPER // GROUP


def _sc_gather_body(idx_hbm, tab_hbm, out_hbm, idx_v, stage1, stage2, sem):
    wid = lax.axis_index("s") * NC + lax.axis_index("c")
    pltpu.sync_copy(idx_hbm.at[pl.ds(wid * (BATCH_PER_W // GROUP),
                                     BATCH_PER_W // GROUP)], idx_v)

    zeros = jnp.zeros((EMB,), jnp.float32)

    def superchunk(s, _):
        b0 = wid * BATCH_PER_W + s * SUPER

        def g_copy(k):
            return pltpu.make_async_copy(
                tab_hbm.at[idx_v.at[s * GPS + k]],
                stage1.at[pl.ds(k * GIDX, GIDX)], sem)

        def fire(k, _):
            g_copy(k).start()
            return _

        lax.fori_loop(0, GPS, fire, None)

        def drain(k, _):
            g_copy(k).wait()
            return _

        lax.fori_loop(0, GPS, drain, None)

        def repack(bl, _):
            base = bl * N_FIELDS
            for p in range(NPLANE):
                for e in range(8):
                    f = 8 * p + e
                    if f < N_FIELDS:
                        stage2[p, bl, pl.ds(EMB * e, EMB)] = stage1[base + f, :]
                    else:
                        stage2[p, bl, pl.ds(EMB * e, EMB)] = zeros
            return _

        lax.fori_loop(0, SUPER, repack, None)

        for p in range(NPLANE):
            pltpu.sync_copy(stage2.at[p], out_hbm.at[p, pl.ds(b0, SUPER)])
        return _

    lax.fori_loop(0, NSUPER, superchunk, None)


_sc_gather = pl.kernel(
    _sc_gather_body,
    out_type=jax.ShapeDtypeStruct((NPLANE, BATCH, HID), jnp.float32),
    mesh=plsc.VectorSubcoreMesh(core_axis_name="c", subcore_axis_name="s"),
    compiler_params=pltpu.CompilerParams(use_tc_tiling_on_sc=False),
    scratch_types=[
        pltpu.VMEM((BATCH_PER_W // GROUP, GIDX), jnp.int32),
        pltpu.VMEM((SUPER * N_FIELDS, EMB), jnp.float32),
        pltpu.VMEM((NPLANE, SUPER, HID), jnp.float32),
        pltpu.SemaphoreType.DMA,
    ],
)


def _mm_body(g_ref, o_ref, w1_ref, w2_ref, b_ref, out_ref):
    acc = jnp.dot(o_ref[...], w2_ref[...], preferred_element_type=jnp.float32)
    for p in range(NPLANE):
        acc += jnp.dot(g_ref[p], w1_ref[p],
                       preferred_element_type=jnp.float32)
    out_ref[...] = acc + b_ref[...]


def _dense(planes, ohes, w1, w2, b2):
    bm = 1024
    return pl.pallas_call(
        _mm_body,
        grid=(BATCH // bm,),
        in_specs=[
            pl.BlockSpec((NPLANE, bm, HID), lambda m: (0, m, 0)),
            pl.BlockSpec((bm, OHE), lambda m: (m, 0)),
            pl.BlockSpec((NPLANE, HID, HID), lambda m: (0, 0, 0)),
            pl.BlockSpec((OHE, HID), lambda m: (0, 0)),
            pl.BlockSpec((1, HID), lambda m: (0, 0)),
        ],
        out_specs=pl.BlockSpec((bm, HID), lambda m: (m, 0)),
        out_shape=jax.ShapeDtypeStruct((BATCH, HID), jnp.float32),
    )(planes, ohes, w1, w2, b2)


def kernel(embed_idx, ohes, tables, W, b):
    offs = (jnp.arange(N_FIELDS, dtype=jnp.int32) * VOCAB)[None, :]
    idx2d = (embed_idx.astype(jnp.int32) + offs).reshape(BATCH // GROUP, GIDX)
    tab_flat = tables.reshape(N_FIELDS * VOCAB, EMB)
    planes = _sc_gather(idx2d, tab_flat)
    w1 = jnp.pad(W[:EMB_FEAT], ((0, NPLANE * HID - EMB_FEAT), (0, 0)))
    w1 = w1.reshape(NPLANE, HID, HID)
    return _dense(planes, ohes, w1, W[EMB_FEAT:], b.reshape(1, HID))


# speedup vs baseline: 1.0014x; 1.0014x over previous
import functools

import jax
import jax.numpy as jnp
from jax import lax
from jax.experimental import pallas as pl
from jax.experimental.pallas import tpu as pltpu
from jax.experimental.pallas import tpu_sc as plsc

N_FIELDS = 26
VOCAB = 100000
EMB = 16
OHE = 100
HID = 128
BATCH = 16384
EMB_FEAT = N_FIELDS * EMB
NPLANE = 4

NC, NS = 2, 16
NW = NC * NS
BATCH_PER_W = BATCH // NW
SUPER = 64
NSUPER = BATCH_PER_W // SUPER


def _sc_gather_body(idx_hbm, tab_hbm, out_hbm, idx_v, stage1, stage2, sem):
    wid = lax.axis_index("s") * NC + lax.axis_index("c")
    wb0 = wid * BATCH_PER_W
    pltpu.sync_copy(idx_hbm.at[:, pl.ds(wb0, BATCH_PER_W)], idx_v)

    zeros = jnp.zeros((EMB,), jnp.float32)

    def superchunk(s, _):
        b0 = s * SUPER

        def g_copy(f):
            return pltpu.make_async_copy(
                tab_hbm.at[f].at[idx_v.at[f, pl.ds(b0, SUPER)]],
                stage1.at[f], sem)

        def fire(f, _):
            g_copy(f).start()
            return _

        lax.fori_loop(0, N_FIELDS, fire, None)

        def drain(f, _):
            g_copy(f).wait()
            return _

        lax.fori_loop(0, N_FIELDS, drain, None)

        def repack(bl, _):
            for p in range(NPLANE):
                for e in range(8):
                    f = 8 * p + e
                    if f < N_FIELDS:
                        stage2[p, bl, pl.ds(EMB * e, EMB)] = stage1[f, bl, :]
                    else:
                        stage2[p, bl, pl.ds(EMB * e, EMB)] = zeros
            return _

        lax.fori_loop(0, SUPER, repack, None)

        for p in range(NPLANE):
            pltpu.sync_copy(stage2.at[p],
                            out_hbm.at[p, pl.ds(wb0 + b0, SUPER)])
        return _

    lax.fori_loop(0, NSUPER, superchunk, None)


_sc_gather = pl.kernel(
    _sc_gather_body,
    out_type=jax.ShapeDtypeStruct((NPLANE, BATCH, HID), jnp.float32),
    mesh=plsc.VectorSubcoreMesh(core_axis_name="c", subcore_axis_name="s"),
    compiler_params=pltpu.CompilerParams(use_tc_tiling_on_sc=False),
    scratch_types=[
        pltpu.VMEM((N_FIELDS, BATCH_PER_W), jnp.int32),
        pltpu.VMEM((N_FIELDS, SUPER, EMB), jnp.float32),
        pltpu.VMEM((NPLANE, SUPER, HID), jnp.float32),
        pltpu.SemaphoreType.DMA,
    ],
)


def _mm_body(g_ref, o_ref, w1_ref, w2_ref, b_ref, out_ref):
    acc = jnp.dot(o_ref[...], w2_ref[...], preferred_element_type=jnp.float32)
    for p in range(NPLANE):
        acc += jnp.dot(g_ref[p], w1_ref[p],
                       preferred_element_type=jnp.float32)
    out_ref[...] = acc + b_ref[...]


def _dense(planes, ohes, w1, w2, b2):
    bm = 1024
    return pl.pallas_call(
        _mm_body,
        grid=(BATCH // bm,),
        in_specs=[
            pl.BlockSpec((NPLANE, bm, HID), lambda m: (0, m, 0)),
            pl.BlockSpec((bm, OHE), lambda m: (m, 0)),
            pl.BlockSpec((NPLANE, HID, HID), lambda m: (0, 0, 0)),
            pl.BlockSpec((OHE, HID), lambda m: (0, 0)),
            pl.BlockSpec((1, HID), lambda m: (0, 0)),
        ],
        out_specs=pl.BlockSpec((bm, HID), lambda m: (m, 0)),
        out_shape=jax.ShapeDtypeStruct((BATCH, HID), jnp.float32),
    )(planes, ohes, w1, w2, b2)


def kernel(embed_idx, ohes, tables, W, b):
    idx_t = embed_idx.astype(jnp.int32).T
    planes = _sc_gather(idx_t, tables)
    w1 = jnp.pad(W[:EMB_FEAT], ((0, NPLANE * HID - EMB_FEAT), (0, 0)))
    w1 = w1.reshape(NPLANE, HID, HID)
    return _dense(planes, ohes, w1, W[EMB_FEAT:], b.reshape(1, HID))
